# Initial kernel scaffold; baseline (speedup 1.0000x reference)
import jax
import jax.numpy as jnp
from jax.experimental import pallas as pl


def kernel(x, W):
    raise NotImplementedError("write your pallas kernel here")


# speedup vs baseline: 1.3285x; 1.3285x over previous
import functools

import jax
import jax.numpy as jnp
from jax.experimental import pallas as pl
from jax.experimental.pallas import tpu as pltpu

NUM_EXPERTS = 64
TOP_K = 2
BLOCK_M = 512


def _router_block(x_ref, w_ref, ids_ref, probs_ref, logits_ref):
    l = jnp.dot(x_ref[...], w_ref[...], preferred_element_type=jnp.float32)
    logits_ref[...] = l
    e = jax.lax.broadcasted_iota(jnp.int32, l.shape, 1)
    m1 = jnp.max(l, axis=-1, keepdims=True)
    i1 = jnp.min(jnp.where(l == m1, e, NUM_EXPERTS), axis=-1, keepdims=True)
    neg = jnp.finfo(jnp.float32).min
    lm = jnp.where(e == i1, neg, l)
    m2 = jnp.max(lm, axis=-1, keepdims=True)
    i2 = jnp.min(jnp.where(lm == m2, e, NUM_EXPERTS), axis=-1, keepdims=True)
    e2 = jnp.exp(m2 - m1)
    p1 = 1.0 / (1.0 + e2)
    p2 = 1.0 - p1
    ids_ref[...] = jnp.concatenate([i1, i2], axis=-1)
    probs_ref[...] = jnp.concatenate([p1, p2], axis=-1)


@jax.jit
def kernel(x, W):
    orig_ndim = x.ndim
    if orig_ndim == 3:
        x = x.reshape(-1, x.shape[-1])
    n, d = x.shape
    num_e = W.shape[1]
    grid = (n // BLOCK_M,)
    ids, probs, logits = pl.pallas_call(
        _router_block,
        grid=grid,
        in_specs=[
            pl.BlockSpec((BLOCK_M, d), lambda i: (i, 0)),
            pl.BlockSpec((d, num_e), lambda i: (0, 0)),
        ],
        out_specs=[
            pl.BlockSpec((BLOCK_M, TOP_K), lambda i: (i, 0)),
            pl.BlockSpec((BLOCK_M, TOP_K), lambda i: (i, 0)),
            pl.BlockSpec((BLOCK_M, num_e), lambda i: (i, 0)),
        ],
        out_shape=[
            jax.ShapeDtypeStruct((n, TOP_K), jnp.int32),
            jax.ShapeDtypeStruct((n, TOP_K), jnp.float32),
            jax.ShapeDtypeStruct((n, num_e), jnp.float32),
        ],
        compiler_params=pltpu.CompilerParams(
            dimension_semantics=("arbitrary",),
        ),
    )(x, W)
    return ids, probs, logits


# speedup vs baseline: 1.3897x; 1.0461x over previous
import functools

import jax
import jax.numpy as jnp
from jax.experimental import pallas as pl
from jax.experimental.pallas import tpu as pltpu

NUM_EXPERTS = 64
TOP_K = 2
BLOCK_M = 1024


def _router_block(x_ref, w_ref, ids_ref, probs_ref, logits_ref):
    l = jnp.dot(x_ref[...], w_ref[...], preferred_element_type=jnp.float32)
    logits_ref[...] = l
    e = jax.lax.broadcasted_iota(jnp.int32, l.shape, 1)
    m1 = jnp.max(l, axis=-1, keepdims=True)
    i1 = jnp.min(jnp.where(l == m1, e, NUM_EXPERTS), axis=-1, keepdims=True)
    neg = jnp.finfo(jnp.float32).min
    lm = jnp.where(e == i1, neg, l)
    m2 = jnp.max(lm, axis=-1, keepdims=True)
    i2 = jnp.min(jnp.where(lm == m2, e, NUM_EXPERTS), axis=-1, keepdims=True)
    e2 = jnp.exp(m2 - m1)
    p1 = 1.0 / (1.0 + e2)
    p2 = 1.0 - p1
    ids_ref[...] = jnp.concatenate([i1, i2], axis=-1)
    probs_ref[...] = jnp.concatenate([p1, p2], axis=-1)


@jax.jit
def kernel(x, W):
    orig_ndim = x.ndim
    if orig_ndim == 3:
        x = x.reshape(-1, x.shape[-1])
    n, d = x.shape
    num_e = W.shape[1]
    grid = (n // BLOCK_M,)
    ids, probs, logits = pl.pallas_call(
        _router_block,
        grid=grid,
        in_specs=[
            pl.BlockSpec((BLOCK_M, d), lambda i: (i, 0)),
            pl.BlockSpec((d, num_e), lambda i: (0, 0)),
        ],
        out_specs=[
            pl.BlockSpec((BLOCK_M, TOP_K), lambda i: (i, 0)),
            pl.BlockSpec((BLOCK_M, TOP_K), lambda i: (i, 0)),
            pl.BlockSpec((BLOCK_M, num_e), lambda i: (i, 0)),
        ],
        out_shape=[
            jax.ShapeDtypeStruct((n, TOP_K), jnp.int32),
            jax.ShapeDtypeStruct((n, TOP_K), jnp.float32),
            jax.ShapeDtypeStruct((n, num_e), jnp.float32),
        ],
        compiler_params=pltpu.CompilerParams(
            dimension_semantics=("arbitrary",),
        ),
    )(x, W)
    return ids, probs, logits


# speedup vs baseline: 1.3913x; 1.0011x over previous
import functools

import jax
import jax.numpy as jnp
from jax.experimental import pallas as pl
from jax.experimental.pallas import tpu as pltpu

NUM_EXPERTS = 64
TOP_K = 2
BLOCK_M = 1024


def _router_block(x_ref, w_ref, ids_ref, probs_ref, logits_ref):
    l = jnp.dot(x_ref[...], w_ref[...], preferred_element_type=jnp.float32)
    logits_ref[...] = l
    e = jax.lax.broadcasted_iota(jnp.int32, l.shape, 1)
    m1 = jnp.max(l, axis=-1, keepdims=True)
    i1 = jnp.min(jnp.where(l == m1, e, NUM_EXPERTS), axis=-1, keepdims=True)
    neg = jnp.finfo(jnp.float32).min
    lm = jnp.where(e == i1, neg, l)
    m2 = jnp.max(lm, axis=-1, keepdims=True)
    i2 = jnp.min(jnp.where(lm == m2, e, NUM_EXPERTS), axis=-1, keepdims=True)
    e2 = jnp.exp(m2 - m1)
    p1 = 1.0 / (1.0 + e2)
    p2 = 1.0 - p1
    ids_ref[...] = jnp.concatenate([i1, i2], axis=-1)
    probs_ref[...] = jnp.concatenate([p1, p2], axis=-1)


@jax.jit
def kernel(x, W):
    orig_ndim = x.ndim
    if orig_ndim == 3:
        x = x.reshape(-1, x.shape[-1])
    n, d = x.shape
    num_e = W.shape[1]
    grid = (n // BLOCK_M,)
    ids, probs, logits = pl.pallas_call(
        _router_block,
        grid=grid,
        in_specs=[
            pl.BlockSpec((BLOCK_M, d), lambda i: (i, 0)),
            pl.BlockSpec((d, num_e), lambda i: (0, 0)),
        ],
        out_specs=[
            pl.BlockSpec((BLOCK_M, TOP_K), lambda i: (i, 0)),
            pl.BlockSpec((BLOCK_M, TOP_K), lambda i: (i, 0)),
            pl.BlockSpec((BLOCK_M, num_e), lambda i: (i, 0)),
        ],
        out_shape=[
            jax.ShapeDtypeStruct((n, TOP_K), jnp.int32),
            jax.ShapeDtypeStruct((n, TOP_K), jnp.float32),
            jax.ShapeDtypeStruct((n, num_e), jnp.float32),
        ],
        compiler_params=pltpu.CompilerParams(
            dimension_semantics=("parallel",),
        ),
    )(x, W)
    return ids, probs, logits
